# Initial kernel scaffold; baseline (speedup 1.0000x reference)
import jax
import jax.numpy as jnp
from jax.experimental import pallas as pl


def kernel(x, router_w, gate_proj, up_proj, down_proj):
    raise NotImplementedError("write your pallas kernel here")


# speedup vs baseline: 2.7595x; 2.7595x over previous
import functools

import jax
import jax.numpy as jnp
from jax import lax
from jax.experimental import pallas as pl
from jax.experimental.pallas import tpu as pltpu
from jax.experimental.pallas import tpu_sc as plsc

T = 2048
D = 1024
F = 1024
E = 8
K = 2
TB = 256
NTB = T // TB
BM = 256
NPAIR = K * T
MPAD = NPAIR + E * BM
NBLK = MPAD // BM
BE_PAD = 128

NC = 2
NS = 16
NW = NC * NS
PPW = NPAIR // NW
DCH = 16
NDCH = PPW // DCH
TPW = T // NW
NCCH = TPW // DCH


def _router_body(x_ref, rw_ref, pos_ref, w_ref, be_ref,
                 p0s, oh0s, p1s, oh1s, w0s, w1s, cnt, tri):
    j = pl.program_id(0)

    @pl.when(j == 0)
    def _init():
        cnt[...] = jnp.zeros((1, E), jnp.float32)
        ri = lax.broadcasted_iota(jnp.int32, (TB, TB), 0)
        ci = lax.broadcasted_iota(jnp.int32, (TB, TB), 1)
        tri[...] = (ri > ci).astype(jnp.float32)

    @pl.when(j < NTB)
    def _block():
        x = x_ref[...]
        scores = jnp.dot(x, rw_ref[...], preferred_element_type=jnp.float32)
        m = jnp.max(scores, axis=1, keepdims=True)
        ex = jnp.exp(scores - m)
        p = ex / jnp.sum(ex, axis=1, keepdims=True)

        iota8 = lax.broadcasted_iota(jnp.int32, (TB, E), 1)
        m1 = jnp.max(p, axis=1, keepdims=True)
        i1 = jnp.min(jnp.where(p == m1, iota8, E), axis=1, keepdims=True)
        oh0 = (iota8 == i1).astype(jnp.float32)
        pm = jnp.where(iota8 == i1, -jnp.inf, p)
        m2 = jnp.max(pm, axis=1, keepdims=True)
        i2 = jnp.min(jnp.where(pm == m2, iota8, E), axis=1, keepdims=True)
        oh1 = (iota8 == i2).astype(jnp.float32)
        s = m1 + m2
        w0 = m1 / s
        w1 = m2 / s

        ltri = tri[...]
        c0 = cnt[...]
        ranks0 = jnp.dot(ltri, oh0, preferred_element_type=jnp.float32) + c0
        c1 = c0 + jnp.sum(oh0, axis=0, keepdims=True)
        ranks1 = jnp.dot(ltri, oh1, preferred_element_type=jnp.float32) + c1
        cnt[...] = c1 + jnp.sum(oh1, axis=0, keepdims=True)

        p0s[j] = ranks0 * oh0
        oh0s[j] = oh0
        p1s[j] = ranks1 * oh1
        oh1s[j] = oh1
        w0s[j] = w0
        w1s[j] = w1

    @pl.when(j == NTB)
    def _final():
        counts = cnt[...]
        ac = jnp.ceil(counts / BM) * BM
        ei = lax.broadcasted_iota(jnp.int32, (E, E), 0)
        ej = lax.broadcasted_iota(jnp.int32, (E, E), 1)
        excl = (ei < ej).astype(jnp.float32)
        incl = (ei <= ej).astype(jnp.float32)
        offs = jnp.dot(ac, excl, preferred_element_type=jnp.float32)
        cb = jnp.dot(ac, incl, preferred_element_type=jnp.float32) / BM

        pos0_cols = []
        pos1_cols = []
        for jj in range(NTB):
            oh0 = oh0s[jj]
            oh1 = oh1s[jj]
            pos0_cols.append(jnp.sum(p0s[jj] + offs * oh0, axis=1,
                                     keepdims=True))
            pos1_cols.append(jnp.sum(p1s[jj] + offs * oh1, axis=1,
                                     keepdims=True))
        pos0 = jnp.concatenate(pos0_cols, axis=1).astype(jnp.int32)
        pos1 = jnp.concatenate(pos1_cols, axis=1).astype(jnp.int32)
        pos_ref[0] = pos0
        pos_ref[1] = pos1
        w_ref[0] = jnp.concatenate([w0s[jj] for jj in range(NTB)], axis=1)
        w_ref[1] = jnp.concatenate([w1s[jj] for jj in range(NTB)], axis=1)

        cbT = jnp.transpose(cb).astype(jnp.int32)
        bi = lax.broadcasted_iota(jnp.int32, (E, BE_PAD), 1)
        be = jnp.sum((bi >= cbT).astype(jnp.int32), axis=0, keepdims=True)
        be_ref[...] = jnp.minimum(be, E - 1)


def _router(x2, router_w):
    return pl.pallas_call(
        _router_body,
        grid=(NTB + 1,),
        in_specs=[
            pl.BlockSpec((TB, D), lambda j: (jnp.minimum(j, NTB - 1), 0)),
            pl.BlockSpec((D, E), lambda j: (0, 0)),
        ],
        out_specs=[
            pl.BlockSpec((K, TB, NTB), lambda j: (0, 0, 0)),
            pl.BlockSpec((K, TB, NTB), lambda j: (0, 0, 0)),
            pl.BlockSpec((1, BE_PAD), lambda j: (0, 0)),
        ],
        out_shape=[
            jax.ShapeDtypeStruct((K, TB, NTB), jnp.int32),
            jax.ShapeDtypeStruct((K, TB, NTB), jnp.float32),
            jax.ShapeDtypeStruct((1, BE_PAD), jnp.int32),
        ],
        scratch_shapes=[
            pltpu.VMEM((NTB, TB, E), jnp.float32),
            pltpu.VMEM((NTB, TB, E), jnp.float32),
            pltpu.VMEM((NTB, TB, E), jnp.float32),
            pltpu.VMEM((NTB, TB, E), jnp.float32),
            pltpu.VMEM((NTB, TB, 1), jnp.float32),
            pltpu.VMEM((NTB, TB, 1), jnp.float32),
            pltpu.VMEM((1, E), jnp.float32),
            pltpu.VMEM((TB, TB), jnp.float32),
        ],
    )(x2, router_w)


def _dispatch_sc(x2, posf, wf):
    mesh = plsc.VectorSubcoreMesh(core_axis_name="c", subcore_axis_name="s")

    @functools.partial(
        pl.kernel, mesh=mesh,
        out_type=[
            jax.ShapeDtypeStruct((MPAD, D), jnp.float32),
            jax.ShapeDtypeStruct((MPAD,), jnp.float32),
        ],
        scratch_types=[
            pltpu.VMEM((NDCH, DCH), jnp.int32),
            pltpu.VMEM((NDCH, DCH), jnp.float32),
            pltpu.VMEM((DCH, D), jnp.float32),
            pltpu.SemaphoreType.DMA,
            pltpu.SemaphoreType.DMA,
            pltpu.SemaphoreType.DMA,
        ],
    )
    def k(x_hbm, pos_hbm, w_hbm, xs_hbm, wrow_hbm,
          posb, wb, rows, sg, ss, sw):
        wid = lax.axis_index("s") * NC + lax.axis_index("c")
        base = wid * PPW
        for j in range(NDCH):
            p0 = base + j * DCH
            pltpu.sync_copy(pos_hbm.at[pl.ds(p0, DCH)], posb.at[j])
            pltpu.sync_copy(w_hbm.at[pl.ds(p0, DCH)], wb.at[j])
            tok = (p0 + lax.broadcasted_iota(jnp.int32, (DCH,), 0)) & (T - 1)
            pltpu.async_copy(x_hbm.at[tok], rows, sg).wait()
            cs = pltpu.async_copy(rows, xs_hbm.at[posb.at[j]], ss)
            cw = pltpu.async_copy(wb.at[j], wrow_hbm.at[posb.at[j]], sw)
            cs.wait()
            cw.wait()

    return k(x2, posf, wf)


def _ffn_body(be_ref, xs_ref, g_ref, u_ref, d_ref, w_ref, o_ref):
    x = xs_ref[...]
    g = jnp.dot(x, g_ref[0], preferred_element_type=jnp.float32)
    u = jnp.dot(x, u_ref[0], preferred_element_type=jnp.float32)
    a = g * jax.nn.sigmoid(g) * u
    o = jnp.dot(a, d_ref[0], preferred_element_type=jnp.float32)
    o_ref[...] = o * w_ref[0]


def _ffn(be_flat, xs, gate_proj, up_proj, down_proj, wrow3):
    grid_spec = pltpu.PrefetchScalarGridSpec(
        num_scalar_prefetch=1,
        grid=(NBLK,),
        in_specs=[
            pl.BlockSpec((BM, D), lambda i, be: (i, 0)),
            pl.BlockSpec((1, D, F), lambda i, be: (be[i], 0, 0)),
            pl.BlockSpec((1, D, F), lambda i, be: (be[i], 0, 0)),
            pl.BlockSpec((1, F, D), lambda i, be: (be[i], 0, 0)),
            pl.BlockSpec((1, BM, 1), lambda i, be: (i, 0, 0)),
        ],
        out_specs=pl.BlockSpec((BM, D), lambda i, be: (i, 0)),
    )
    return pl.pallas_call(
        _ffn_body,
        grid_spec=grid_spec,
        out_shape=jax.ShapeDtypeStruct((MPAD, D), jnp.float32),
    )(be_flat, xs, gate_proj, up_proj, down_proj, wrow3)


def _combine_sc(ys, posf):
    mesh = plsc.VectorSubcoreMesh(core_axis_name="c", subcore_axis_name="s")

    @functools.partial(
        pl.kernel, mesh=mesh,
        out_type=jax.ShapeDtypeStruct((T, D), jnp.float32),
        scratch_types=[
            pltpu.VMEM((NCCH, DCH), jnp.int32),
            pltpu.VMEM((NCCH, DCH), jnp.int32),
            pltpu.VMEM((DCH, D), jnp.float32),
            pltpu.VMEM((DCH, D), jnp.float32),
            pltpu.SemaphoreType.DMA,
            pltpu.SemaphoreType.DMA,
        ],
    )
    def k(ys_hbm, pos_hbm, y_hbm, i0b, i1b, buf0, buf1, s0, s1):
        wid = lax.axis_index("s") * NC + lax.axis_index("c")
        base = wid * TPW
        for j in range(NCCH):
            t0 = base + j * DCH
            pltpu.sync_copy(pos_hbm.at[pl.ds(t0, DCH)], i0b.at[j])
            pltpu.sync_copy(pos_hbm.at[pl.ds(T + t0, DCH)], i1b.at[j])
            c0 = pltpu.async_copy(ys_hbm.at[i0b.at[j]], buf0, s0)
            c1 = pltpu.async_copy(ys_hbm.at[i1b.at[j]], buf1, s1)
            c0.wait()
            c1.wait()

            def add_col(c, _):
                for r in range(DCH):
                    sl = pl.ds(c * 16, 16)
                    buf0[r, sl] = buf0[r, sl] + buf1[r, sl]
                return 0

            lax.fori_loop(0, D // 16, add_col, 0)
            pltpu.sync_copy(buf0, y_hbm.at[pl.ds(t0, DCH)])

    return k(ys, posf)


def kernel(x, router_w, gate_proj, up_proj, down_proj):
    x2 = x.reshape(T, D)
    pos_b, w_b, be = _router(x2, router_w)
    posf = pos_b.transpose(0, 2, 1).reshape(NPAIR)
    wf = w_b.transpose(0, 2, 1).reshape(NPAIR)
    be_flat = be.reshape(BE_PAD)
    xs, wrow = _dispatch_sc(x2, posf, wf)
    wrow3 = wrow.reshape(NBLK, BM, 1)
    ys = _ffn(be_flat, xs, gate_proj, up_proj, down_proj, wrow3)
    y = _combine_sc(ys, posf)
    return y.reshape(1, T, D)


# speedup vs baseline: 2.7610x; 1.0005x over previous
import functools

import jax
import jax.numpy as jnp
from jax import lax
from jax.experimental import pallas as pl
from jax.experimental.pallas import tpu as pltpu
from jax.experimental.pallas import tpu_sc as plsc

T = 2048
D = 1024
F = 1024
E = 8
K = 2
TB = 256
NTB = T // TB
BM = 256
NPAIR = K * T
MPAD = NPAIR + E * BM
NBLK = MPAD // BM
BE_PAD = 128

NC = 2
NS = 16
NW = NC * NS
PPW = NPAIR // NW
DCH = 16
NDCH = PPW // DCH
TPW = T // NW
NCCH = TPW // DCH


def _router_body(x_ref, rw_ref, pos_ref, w_ref, be_ref,
                 p0s, oh0s, p1s, oh1s, w0s, w1s, cnt, tri):
    j = pl.program_id(0)

    @pl.when(j == 0)
    def _init():
        cnt[...] = jnp.zeros((1, E), jnp.float32)
        ri = lax.broadcasted_iota(jnp.int32, (TB, TB), 0)
        ci = lax.broadcasted_iota(jnp.int32, (TB, TB), 1)
        tri[...] = (ri > ci).astype(jnp.float32)

    @pl.when(j < NTB)
    def _block():
        x = x_ref[...]
        scores = jnp.dot(x, rw_ref[...], preferred_element_type=jnp.float32)
        m = jnp.max(scores, axis=1, keepdims=True)
        ex = jnp.exp(scores - m)
        p = ex / jnp.sum(ex, axis=1, keepdims=True)

        iota8 = lax.broadcasted_iota(jnp.int32, (TB, E), 1)
        m1 = jnp.max(p, axis=1, keepdims=True)
        i1 = jnp.min(jnp.where(p == m1, iota8, E), axis=1, keepdims=True)
        oh0 = (iota8 == i1).astype(jnp.float32)
        pm = jnp.where(iota8 == i1, -jnp.inf, p)
        m2 = jnp.max(pm, axis=1, keepdims=True)
        i2 = jnp.min(jnp.where(pm == m2, iota8, E), axis=1, keepdims=True)
        oh1 = (iota8 == i2).astype(jnp.float32)
        s = m1 + m2
        w0 = m1 / s
        w1 = m2 / s

        ltri = tri[...]
        c0 = cnt[...]
        ranks0 = jnp.dot(ltri, oh0, preferred_element_type=jnp.float32) + c0
        c1 = c0 + jnp.sum(oh0, axis=0, keepdims=True)
        ranks1 = jnp.dot(ltri, oh1, preferred_element_type=jnp.float32) + c1
        cnt[...] = c1 + jnp.sum(oh1, axis=0, keepdims=True)

        p0s[j] = ranks0 * oh0
        oh0s[j] = oh0
        p1s[j] = ranks1 * oh1
        oh1s[j] = oh1
        w0s[j] = w0
        w1s[j] = w1

    @pl.when(j == NTB)
    def _final():
        counts = cnt[...]
        ac = jnp.ceil(counts / BM) * BM
        ei = lax.broadcasted_iota(jnp.int32, (E, E), 0)
        ej = lax.broadcasted_iota(jnp.int32, (E, E), 1)
        excl = (ei < ej).astype(jnp.float32)
        incl = (ei <= ej).astype(jnp.float32)
        offs = jnp.dot(ac, excl, preferred_element_type=jnp.float32)
        cb = jnp.dot(ac, incl, preferred_element_type=jnp.float32) / BM

        pos0_cols = []
        pos1_cols = []
        for jj in range(NTB):
            oh0 = oh0s[jj]
            oh1 = oh1s[jj]
            pos0_cols.append(jnp.sum(p0s[jj] + offs * oh0, axis=1,
                                     keepdims=True))
            pos1_cols.append(jnp.sum(p1s[jj] + offs * oh1, axis=1,
                                     keepdims=True))
        pos0 = jnp.concatenate(pos0_cols, axis=1).astype(jnp.int32)
        pos1 = jnp.concatenate(pos1_cols, axis=1).astype(jnp.int32)
        pos_ref[0] = pos0
        pos_ref[1] = pos1
        w_ref[0] = jnp.concatenate([w0s[jj] for jj in range(NTB)], axis=1)
        w_ref[1] = jnp.concatenate([w1s[jj] for jj in range(NTB)], axis=1)

        cbT = jnp.transpose(cb).astype(jnp.int32)
        bi = lax.broadcasted_iota(jnp.int32, (E, BE_PAD), 1)
        be = jnp.sum((bi >= cbT).astype(jnp.int32), axis=0, keepdims=True)
        be_ref[...] = jnp.minimum(be, E - 1)


def _router(x2, router_w):
    return pl.pallas_call(
        _router_body,
        grid=(NTB + 1,),
        in_specs=[
            pl.BlockSpec((TB, D), lambda j: (jnp.minimum(j, NTB - 1), 0)),
            pl.BlockSpec((D, E), lambda j: (0, 0)),
        ],
        out_specs=[
            pl.BlockSpec((K, TB, NTB), lambda j: (0, 0, 0)),
            pl.BlockSpec((K, TB, NTB), lambda j: (0, 0, 0)),
            pl.BlockSpec((1, BE_PAD), lambda j: (0, 0)),
        ],
        out_shape=[
            jax.ShapeDtypeStruct((K, TB, NTB), jnp.int32),
            jax.ShapeDtypeStruct((K, TB, NTB), jnp.float32),
            jax.ShapeDtypeStruct((1, BE_PAD), jnp.int32),
        ],
        scratch_shapes=[
            pltpu.VMEM((NTB, TB, E), jnp.float32),
            pltpu.VMEM((NTB, TB, E), jnp.float32),
            pltpu.VMEM((NTB, TB, E), jnp.float32),
            pltpu.VMEM((NTB, TB, E), jnp.float32),
            pltpu.VMEM((NTB, TB, 1), jnp.float32),
            pltpu.VMEM((NTB, TB, 1), jnp.float32),
            pltpu.VMEM((1, E), jnp.float32),
            pltpu.VMEM((TB, TB), jnp.float32),
        ],
    )(x2, router_w)


def _dispatch_sc(x2, posf, wf):
    mesh = plsc.VectorSubcoreMesh(core_axis_name="c", subcore_axis_name="s")

    @functools.partial(
        pl.kernel, mesh=mesh,
        out_type=[
            jax.ShapeDtypeStruct((MPAD, D), jnp.float32),
            jax.ShapeDtypeStruct((MPAD,), jnp.float32),
        ],
        scratch_types=[
            pltpu.VMEM((NDCH, DCH), jnp.int32),
            pltpu.VMEM((NDCH, DCH), jnp.float32),
            pltpu.VMEM((DCH, D), jnp.float32),
            pltpu.SemaphoreType.DMA,
            pltpu.SemaphoreType.DMA,
            pltpu.SemaphoreType.DMA,
        ],
    )
    def k(x_hbm, pos_hbm, w_hbm, xs_hbm, wrow_hbm,
          posb, wb, rows, sg, ss, sw):
        wid = lax.axis_index("s") * NC + lax.axis_index("c")
        base = wid * PPW
        for j in range(NDCH):
            p0 = base + j * DCH
            pltpu.sync_copy(pos_hbm.at[pl.ds(p0, DCH)], posb.at[j])
            pltpu.sync_copy(w_hbm.at[pl.ds(p0, DCH)], wb.at[j])
            tok = (p0 + lax.broadcasted_iota(jnp.int32, (DCH,), 0)) & (T - 1)
            pltpu.async_copy(x_hbm.at[tok], rows, sg).wait()
            cs = pltpu.async_copy(rows, xs_hbm.at[posb.at[j]], ss)
            cw = pltpu.async_copy(wb.at[j], wrow_hbm.at[posb.at[j]], sw)
            cs.wait()
            cw.wait()

    return k(x2, posf, wf)


def _ffn_body(be_ref, xs_ref, g_ref, u_ref, d_ref, w_ref, o_ref):
    x = xs_ref[...].astype(jnp.bfloat16)
    g = jnp.dot(x, g_ref[0].astype(jnp.bfloat16),
                preferred_element_type=jnp.float32)
    u = jnp.dot(x, u_ref[0].astype(jnp.bfloat16),
                preferred_element_type=jnp.float32)
    a = (g * jax.nn.sigmoid(g) * u).astype(jnp.bfloat16)
    o = jnp.dot(a, d_ref[0].astype(jnp.bfloat16),
                preferred_element_type=jnp.float32)
    o_ref[...] = o * w_ref[0]


def _ffn(be_flat, xs, gate_proj, up_proj, down_proj, wrow3):
    grid_spec = pltpu.PrefetchScalarGridSpec(
        num_scalar_prefetch=1,
        grid=(NBLK,),
        in_specs=[
            pl.BlockSpec((BM, D), lambda i, be: (i, 0)),
            pl.BlockSpec((1, D, F), lambda i, be: (be[i], 0, 0)),
            pl.BlockSpec((1, D, F), lambda i, be: (be[i], 0, 0)),
            pl.BlockSpec((1, F, D), lambda i, be: (be[i], 0, 0)),
            pl.BlockSpec((1, BM, 1), lambda i, be: (i, 0, 0)),
        ],
        out_specs=pl.BlockSpec((BM, D), lambda i, be: (i, 0)),
    )
    return pl.pallas_call(
        _ffn_body,
        grid_spec=grid_spec,
        out_shape=jax.ShapeDtypeStruct((MPAD, D), jnp.float32),
    )(be_flat, xs, gate_proj, up_proj, down_proj, wrow3)


def _combine_sc(ys, posf):
    mesh = plsc.VectorSubcoreMesh(core_axis_name="c", subcore_axis_name="s")

    @functools.partial(
        pl.kernel, mesh=mesh,
        out_type=jax.ShapeDtypeStruct((T, D), jnp.float32),
        scratch_types=[
            pltpu.VMEM((NCCH, DCH), jnp.int32),
            pltpu.VMEM((NCCH, DCH), jnp.int32),
            pltpu.VMEM((DCH, D), jnp.float32),
            pltpu.VMEM((DCH, D), jnp.float32),
            pltpu.SemaphoreType.DMA,
            pltpu.SemaphoreType.DMA,
        ],
    )
    def k(ys_hbm, pos_hbm, y_hbm, i0b, i1b, buf0, buf1, s0, s1):
        wid = lax.axis_index("s") * NC + lax.axis_index("c")
        base = wid * TPW
        for j in range(NCCH):
            t0 = base + j * DCH
            pltpu.sync_copy(pos_hbm.at[pl.ds(t0, DCH)], i0b.at[j])
            pltpu.sync_copy(pos_hbm.at[pl.ds(T + t0, DCH)], i1b.at[j])
            c0 = pltpu.async_copy(ys_hbm.at[i0b.at[j]], buf0, s0)
            c1 = pltpu.async_copy(ys_hbm.at[i1b.at[j]], buf1, s1)
            c0.wait()
            c1.wait()

            def add_col(c, _):
                for r in range(DCH):
                    sl = pl.ds(c * 16, 16)
                    buf0[r, sl] = buf0[r, sl] + buf1[r, sl]
                return 0

            lax.fori_loop(0, D // 16, add_col, 0)
            pltpu.sync_copy(buf0, y_hbm.at[pl.ds(t0, DCH)])

    return k(ys, posf)


def kernel(x, router_w, gate_proj, up_proj, down_proj):
    x2 = x.reshape(T, D)
    pos_b, w_b, be = _router(x2, router_w)
    posf = pos_b.transpose(0, 2, 1).reshape(NPAIR)
    wf = w_b.transpose(0, 2, 1).reshape(NPAIR)
    be_flat = be.reshape(BE_PAD)
    xs, wrow = _dispatch_sc(x2, posf, wf)
    wrow3 = wrow.reshape(NBLK, BM, 1)
    ys = _ffn(be_flat, xs, gate_proj, up_proj, down_proj, wrow3)
    y = _combine_sc(ys, posf)
    return y.reshape(1, T, D)
